# baseline (speedup 1.0000x reference)
import jax
import jax.numpy as jnp
from jax.experimental import pallas as pl
from jax.experimental.pallas import tpu as pltpu

_CHUNKS = (8000, 24000, 24000, 24000, 12000, 8000)
_MAXC = max(_CHUNKS)
_NBUF = 2


def _body(x_hbm, w_ref, o_hbm, xbuf, obuf, insem, outsem):
    offs = []
    o = 0
    for c in _CHUNKS:
        offs.append(o)
        o += c
    n = len(_CHUNKS)

    def in_copy(i):
        return pltpu.make_async_copy(
            x_hbm.at[pl.ds(offs[i], _CHUNKS[i]), :],
            xbuf.at[i % _NBUF, pl.ds(0, _CHUNKS[i]), :],
            insem.at[i % _NBUF])

    def out_copy(i):
        return pltpu.make_async_copy(
            obuf.at[i % _NBUF, pl.ds(0, _CHUNKS[i]), :],
            o_hbm.at[pl.ds(offs[i], _CHUNKS[i]), :],
            outsem.at[i % _NBUF])

    for s in range(_NBUF):
        in_copy(s).start()
    for i in range(n):
        slot = i % _NBUF
        in_copy(i).wait()
        if i >= _NBUF:
            out_copy(i - _NBUF).wait()
        obuf[slot, pl.ds(0, _CHUNKS[i]), :] = jax.lax.dot_general(
            xbuf[slot, pl.ds(0, _CHUNKS[i]), :], w_ref[...],
            dimension_numbers=(((1,), (1,)), ((), ())),
            preferred_element_type=jnp.float32)
        out_copy(i).start()
        if i + _NBUF < n:
            in_copy(i + _NBUF).start()
    for i in range(n - _NBUF, n):
        out_copy(i).wait()


def kernel(x_src, W):
    n, in_ch = x_src.shape
    out_ch = W.shape[0]
    return pl.pallas_call(
        _body,
        in_specs=[
            pl.BlockSpec(memory_space=pl.ANY),
            pl.BlockSpec((out_ch, in_ch), lambda: (0, 0)),
        ],
        out_specs=pl.BlockSpec(memory_space=pl.ANY),
        out_shape=jax.ShapeDtypeStruct((n, out_ch), jnp.float32),
        compiler_params=pltpu.CompilerParams(vmem_limit_bytes=120 * 1024 * 1024),
        scratch_shapes=[
            pltpu.VMEM((_NBUF, _MAXC, in_ch), jnp.float32),
            pltpu.VMEM((_NBUF, _MAXC, out_ch), jnp.float32),
            pltpu.SemaphoreType.DMA((_NBUF,)),
            pltpu.SemaphoreType.DMA((_NBUF,)),
        ],
    )(x_src, W)


# speedup vs baseline: 1.0194x; 1.0194x over previous
import jax
import jax.numpy as jnp
from jax.experimental import pallas as pl
from jax.experimental.pallas import tpu as pltpu

_CHUNKS = (12000, 20000, 20000, 20000, 20000, 8000)
_MAXC = max(_CHUNKS)
_NBUF = 2


def _body(x_hbm, w_ref, o_hbm, xbuf, obuf, insem, outsem):
    offs = []
    o = 0
    for c in _CHUNKS:
        offs.append(o)
        o += c
    n = len(_CHUNKS)

    def in_copy(i):
        return pltpu.make_async_copy(
            x_hbm.at[pl.ds(offs[i], _CHUNKS[i]), :],
            xbuf.at[i % _NBUF, pl.ds(0, _CHUNKS[i]), :],
            insem.at[i % _NBUF])

    def out_copy(i):
        return pltpu.make_async_copy(
            obuf.at[i % _NBUF, pl.ds(0, _CHUNKS[i]), :],
            o_hbm.at[pl.ds(offs[i], _CHUNKS[i]), :],
            outsem.at[i % _NBUF])

    for s in range(_NBUF):
        in_copy(s).start()
    for i in range(n):
        slot = i % _NBUF
        in_copy(i).wait()
        if i >= _NBUF:
            out_copy(i - _NBUF).wait()
        obuf[slot, pl.ds(0, _CHUNKS[i]), :] = jax.lax.dot_general(
            xbuf[slot, pl.ds(0, _CHUNKS[i]), :], w_ref[...],
            dimension_numbers=(((1,), (1,)), ((), ())),
            preferred_element_type=jnp.float32)
        out_copy(i).start()
        if i + _NBUF < n:
            in_copy(i + _NBUF).start()
    for i in range(n - _NBUF, n):
        out_copy(i).wait()


def kernel(x_src, W):
    n, in_ch = x_src.shape
    out_ch = W.shape[0]
    return pl.pallas_call(
        _body,
        in_specs=[
            pl.BlockSpec(memory_space=pl.ANY),
            pl.BlockSpec((out_ch, in_ch), lambda: (0, 0)),
        ],
        out_specs=pl.BlockSpec(memory_space=pl.ANY),
        out_shape=jax.ShapeDtypeStruct((n, out_ch), jnp.float32),
        compiler_params=pltpu.CompilerParams(vmem_limit_bytes=120 * 1024 * 1024),
        scratch_shapes=[
            pltpu.VMEM((_NBUF, _MAXC, in_ch), jnp.float32),
            pltpu.VMEM((_NBUF, _MAXC, out_ch), jnp.float32),
            pltpu.SemaphoreType.DMA((_NBUF,)),
            pltpu.SemaphoreType.DMA((_NBUF,)),
        ],
    )(x_src, W)
